# Initial kernel scaffold; baseline (speedup 1.0000x reference)
import jax
import jax.numpy as jnp
from jax.experimental import pallas as pl


def kernel(hidden, target, cluster_weight, cluster_bias, proj0, proj1, proj2, w0, w1, w2, b0, b1, b2):
    raise NotImplementedError("write your pallas kernel here")


# speedup vs baseline: 1.4572x; 1.4572x over previous
import functools

import jax
import jax.numpy as jnp
from jax.experimental import pallas as pl
from jax.experimental.pallas import tpu as pltpu

_N = 2048
_D = 1024
_CUT1 = 20000
_CUT2 = 40000
_NTOK = 100000
_HEAD = 20002
_NEG = -1e30


def _proj_kernel(h_ref, p_ref, o_ref):
    o_ref[...] = jax.lax.dot_general(
        h_ref[...], p_ref[...], (((1,), (0,)), ((), ())),
        preferred_element_type=jnp.float32).astype(jnp.bfloat16)


def _lse_kernel(idx_ref, hp_ref, wt_ref, b_ref, lse_ref, g_ref,
                m_sc, s_sc, g_sc, *, tile, nsteps):
    i = pl.program_id(0)

    @pl.when(i == 0)
    def _init():
        m_sc[...] = jnp.full_like(m_sc, _NEG)
        s_sc[...] = jnp.zeros_like(s_sc)
        g_sc[...] = jnp.zeros_like(g_sc)

    logits = jax.lax.dot_general(
        hp_ref[...], wt_ref[...], (((1,), (0,)), ((), ())),
        preferred_element_type=jnp.float32)
    logits = logits + b_ref[...]

    cols = jax.lax.broadcasted_iota(jnp.int32, logits.shape, 1) + i * tile
    hit = cols == idx_ref[...]
    g_sc[...] += jnp.sum(jnp.where(hit, logits, 0.0), axis=1, keepdims=True)

    m_old = m_sc[...]
    m_new = jnp.maximum(m_old, jnp.max(logits, axis=1, keepdims=True))
    s_sc[...] = (s_sc[...] * jnp.exp(m_old - m_new)
                 + jnp.sum(jnp.exp(logits - m_new), axis=1, keepdims=True))
    m_sc[...] = m_new

    @pl.when(i == nsteps - 1)
    def _fin():
        lse_ref[...] = jnp.log(s_sc[...]) + m_sc[...]
        g_ref[...] = g_sc[...]


def _combine_kernel(t_ref, lh_ref, gh_ref, l1_ref, g1_ref, l2_ref, g2_ref,
                    o_ref):
    t = t_ref[...]
    nll = lh_ref[...] - gh_ref[...]
    tail1 = l1_ref[...] - g1_ref[...]
    tail2 = l2_ref[...] - g2_ref[...]
    in1 = (t >= _CUT1) & (t < _CUT2)
    in2 = t >= _CUT2
    nll = nll + jnp.where(in1, tail1, 0.0) + jnp.where(in2, tail2, 0.0)
    o_ref[...] = nll


def _stream_lse(hp, wt, b, idx, tile):
    d = hp.shape[1]
    vp = wt.shape[1]
    nsteps = vp // tile
    lse, g = pl.pallas_call(
        functools.partial(_lse_kernel, tile=tile, nsteps=nsteps),
        grid=(nsteps,),
        in_specs=[
            pl.BlockSpec((_N, 1), lambda i: (0, 0)),
            pl.BlockSpec((_N, d), lambda i: (0, 0)),
            pl.BlockSpec((d, tile), lambda i: (0, i)),
            pl.BlockSpec((1, tile), lambda i: (0, i)),
        ],
        out_specs=[
            pl.BlockSpec((_N, 1), lambda i: (0, 0)),
            pl.BlockSpec((_N, 1), lambda i: (0, 0)),
        ],
        out_shape=[
            jax.ShapeDtypeStruct((_N, 1), jnp.float32),
            jax.ShapeDtypeStruct((_N, 1), jnp.float32),
        ],
        scratch_shapes=[
            pltpu.VMEM((_N, 1), jnp.float32),
            pltpu.VMEM((_N, 1), jnp.float32),
            pltpu.VMEM((_N, 1), jnp.float32),
        ],
    )(idx, hp, wt, b)
    return lse, g


def _pad_v(wt, b, tile):
    v = wt.shape[1]
    vp = ((v + tile - 1) // tile) * tile
    if vp != v:
        wt = jnp.pad(wt, ((0, 0), (0, vp - v)))
        b = jnp.pad(b, (0, vp - v), constant_values=_NEG)
    return wt, b.reshape(1, vp).astype(jnp.float32)


def kernel(hidden, target, cluster_weight, cluster_bias, proj0, proj1,
           proj2, w0, w1, w2, b0, b1, b2):
    bf = jnp.bfloat16

    head_wt = jnp.concatenate([w0, cluster_weight], axis=0).T.astype(bf)
    head_b = jnp.concatenate([b0, cluster_bias], axis=0)
    wt1 = w1.T.astype(bf)
    wt2 = w2.T.astype(bf)

    t_head = 512
    t1 = 1024
    t2 = 2048
    head_wt, head_b = _pad_v(head_wt, head_b, t_head)
    wt1, b1p = _pad_v(wt1, b1, t1)
    wt2, b2p = _pad_v(wt2, b2, t2)

    pcat = jnp.concatenate([proj0, proj1, proj2], axis=1).astype(bf)

    hp = pl.pallas_call(
        _proj_kernel,
        out_shape=jax.ShapeDtypeStruct((_N, pcat.shape[1]), bf),
    )(hidden.astype(bf), pcat)
    hp0 = hp[:, :_D]
    hp1 = hp[:, _D:_D + 256]
    hp2 = hp[:, _D + 256:_D + 320]

    t = target.astype(jnp.int32).reshape(_N, 1)
    idx_h = jnp.where(t < _CUT1, t, jnp.where(t < _CUT2, _HEAD - 1, _HEAD - 2))
    idx_1 = jnp.clip(t - _CUT1, 0, _CUT2 - _CUT1 - 1)
    idx_2 = jnp.clip(t - _CUT2, 0, _NTOK - _CUT2 - 1)

    lse_h, g_h = _stream_lse(hp0, head_wt, head_b, idx_h, t_head)
    lse_1, g_1 = _stream_lse(hp1, wt1, b1p, idx_1, t1)
    lse_2, g_2 = _stream_lse(hp2, wt2, b2p, idx_2, t2)

    nll = pl.pallas_call(
        _combine_kernel,
        out_shape=jax.ShapeDtypeStruct((_N, 1), jnp.float32),
    )(t, lse_h, g_h, lse_1, g_1, lse_2, g_2)
    return nll.reshape(_N)


# speedup vs baseline: 1.9948x; 1.3689x over previous
import functools

import jax
import jax.numpy as jnp
from jax.experimental import pallas as pl
from jax.experimental.pallas import tpu as pltpu

_N = 2048
_D = 1024
_CUT1 = 20000
_CUT2 = 40000
_NTOK = 100000
_HEAD = 20002
_NEG = -1e30


def _proj_kernel(p_ref, h_ref, o_ref):
    o_ref[...] = jax.lax.dot_general(
        p_ref[...], h_ref[...], (((1,), (0,)), ((), ())),
        preferred_element_type=jnp.float32).astype(jnp.bfloat16)


def _lse_kernel(idx_ref, hpt_ref, w_ref, b_ref, lse_ref, g_ref,
                iota_sc, m_sc, s_sc, g_sc, *, tile, chunk, nsteps):
    i = pl.program_id(0)
    nch = tile // chunk

    @pl.when(i == 0)
    def _init():
        iota_sc[...] = jax.lax.broadcasted_iota(
            jnp.int32, iota_sc.shape, 0)

    idx = idx_ref[...]
    m_parts, s_parts, g_parts = [], [], []
    for c in range(nch):
        rows = pl.ds(c * chunk, chunk)
        lt = jax.lax.dot_general(
            w_ref[rows, :], hpt_ref[...], (((1,), (0,)), ((), ())),
            preferred_element_type=jnp.float32).astype(jnp.bfloat16)
        lt = lt + b_ref[rows, :]
        hit = iota_sc[...] == (idx - (i * tile + c * chunk))
        g_parts.append(jnp.sum(jnp.where(hit, lt, jnp.bfloat16(0)),
                               axis=0, keepdims=True, dtype=jnp.float32))
        m_c = jnp.max(lt, axis=0, keepdims=True)
        p = jnp.exp(lt - m_c)
        s_parts.append(jnp.sum(p, axis=0, keepdims=True,
                               dtype=jnp.float32))
        m_parts.append(m_c.astype(jnp.float32))
    m_sc[i] = jnp.concatenate(m_parts, axis=0)
    s_sc[i] = jnp.concatenate(s_parts, axis=0)
    g_sc[i] = jnp.concatenate(g_parts, axis=0)

    @pl.when(i == nsteps - 1)
    def _fin():
        m = m_sc[...]
        mm = jnp.max(m, axis=(0, 1), keepdims=True)
        s = jnp.sum(s_sc[...] * jnp.exp(m - mm), axis=(0, 1),
                    keepdims=True)
        lse_ref[...] = (jnp.log(s) + mm).reshape(1, _N)
        g_ref[...] = jnp.sum(g_sc[...], axis=(0, 1), keepdims=True
                             ).reshape(1, _N)


def _combine_kernel(t_ref, lh_ref, gh_ref, l1_ref, g1_ref, l2_ref, g2_ref,
                    o_ref):
    t = t_ref[...]
    nll = lh_ref[...] - gh_ref[...]
    in1 = (t >= _CUT1) & (t < _CUT2)
    in2 = t >= _CUT2
    nll = nll + jnp.where(in1, l1_ref[...] - g1_ref[...], 0.0)
    nll = nll + jnp.where(in2, l2_ref[...] - g2_ref[...], 0.0)
    o_ref[...] = nll


def _stream_lse(hpt, w, b, idx, tile, chunk):
    vp, d = w.shape
    nsteps = vp // tile
    nch = tile // chunk
    vec = jax.ShapeDtypeStruct((1, _N), jnp.float32)
    full = pl.BlockSpec((1, _N), lambda i: (0, 0))
    part = pltpu.VMEM((nsteps, nch, _N), jnp.float32)
    return pl.pallas_call(
        functools.partial(_lse_kernel, tile=tile, chunk=chunk,
                          nsteps=nsteps),
        grid=(nsteps,),
        in_specs=[
            pl.BlockSpec((1, _N), lambda i: (0, 0)),
            pl.BlockSpec((d, _N), lambda i: (0, 0)),
            pl.BlockSpec((tile, d), lambda i: (i, 0)),
            pl.BlockSpec((tile, 1), lambda i: (i, 0)),
        ],
        out_specs=[full, full],
        out_shape=[vec, vec],
        scratch_shapes=[
            pltpu.VMEM((chunk, _N), jnp.int32), part, part, part,
        ],
    )(idx, hpt, w, b)


def _pad_v(w, b, tile):
    v = w.shape[0]
    vp = ((v + tile - 1) // tile) * tile
    if vp != v:
        w = jnp.pad(w, ((0, vp - v), (0, 0)))
        b = jnp.pad(b, (0, vp - v), constant_values=_NEG)
    return w.astype(jnp.bfloat16), b.reshape(vp, 1).astype(jnp.bfloat16)


def kernel(hidden, target, cluster_weight, cluster_bias, proj0, proj1,
           proj2, w0, w1, w2, b0, b1, b2):
    bf = jnp.bfloat16
    tile, chunk = 2048, 512

    w_h, b_h = _pad_v(jnp.concatenate([w0, cluster_weight], axis=0),
                      jnp.concatenate([b0, cluster_bias], axis=0), tile)
    w_1, b_1 = _pad_v(w1, b1, tile)
    w_2, b_2 = _pad_v(w2, b2, tile)

    pt = jnp.concatenate([proj0, proj1, proj2], axis=1).T.astype(bf)
    ht = hidden.T.astype(bf)

    hpt = pl.pallas_call(
        _proj_kernel,
        out_shape=jax.ShapeDtypeStruct((pt.shape[0], _N), bf),
    )(pt, ht)
    hpt0 = hpt[:_D]
    hpt1 = hpt[_D:_D + 256]
    hpt2 = hpt[_D + 256:_D + 320]

    t = target.astype(jnp.int32).reshape(1, _N)
    idx_h = jnp.where(t < _CUT1, t, jnp.where(t < _CUT2, _HEAD - 1, _HEAD - 2))
    idx_1 = jnp.clip(t - _CUT1, 0, _CUT2 - _CUT1 - 1)
    idx_2 = jnp.clip(t - _CUT2, 0, _NTOK - _CUT2 - 1)

    lse_h, g_h = _stream_lse(hpt0, w_h, b_h, idx_h, tile, chunk)
    lse_1, g_1 = _stream_lse(hpt1, w_1, b_1, idx_1, tile, chunk)
    lse_2, g_2 = _stream_lse(hpt2, w_2, b_2, idx_2, tile, chunk)

    nll = pl.pallas_call(
        _combine_kernel,
        out_shape=jax.ShapeDtypeStruct((1, _N), jnp.float32),
    )(t, lse_h, g_h, lse_1, g_1, lse_2, g_2)
    return nll.reshape(_N)


# speedup vs baseline: 2.1416x; 1.0736x over previous
import functools

import jax
import jax.numpy as jnp
from jax.experimental import pallas as pl
from jax.experimental.pallas import tpu as pltpu

_N = 2048
_D = 1024
_CUT1 = 20000
_CUT2 = 40000
_NEG = -1e30


def _proj_kernel(p_ref, h_ref, o_ref):
    o_ref[...] = jax.lax.dot_general(
        p_ref[...], h_ref[...], (((1,), (0,)), ((), ())),
        preferred_element_type=jnp.float32).astype(jnp.bfloat16)


def _lse_kernel(idx_ref, hpt_ref, w_ref, b_ref, lse_ref, g_ref,
                iota_sc, m_sc, s_sc, g_sc, *, tile, chunk, nsteps, nvalid):
    i = pl.program_id(0)
    nch = tile // chunk

    @pl.when(i == 0)
    def _init():
        iota_sc[...] = jax.lax.broadcasted_iota(
            jnp.int32, iota_sc.shape, 0)

    idx = idx_ref[...]
    m_parts, s_parts, g_parts = [], [], []
    for c in range(nch):
        rows = pl.ds(c * chunk, chunk)
        lt = jax.lax.dot_general(
            w_ref[rows, :].astype(jnp.bfloat16), hpt_ref[...],
            (((1,), (0,)), ((), ())),
            preferred_element_type=jnp.float32).astype(jnp.bfloat16)
        lt = lt + b_ref[rows, :].astype(jnp.bfloat16)
        rowid = iota_sc[...] + (i * tile + c * chunk)
        if nvalid % tile != 0:
            lt = jnp.where(rowid < nvalid, lt, jnp.bfloat16(_NEG))
        hit = rowid == idx
        g_parts.append(jnp.sum(jnp.where(hit, lt, jnp.bfloat16(0)),
                               axis=0, keepdims=True, dtype=jnp.float32))
        m_c = jnp.max(lt, axis=0, keepdims=True)
        p = jnp.exp(lt - m_c)
        s_parts.append(jnp.sum(p, axis=0, keepdims=True,
                               dtype=jnp.float32))
        m_parts.append(m_c.astype(jnp.float32))
    m_sc[i] = jnp.concatenate(m_parts, axis=0)
    s_sc[i] = jnp.concatenate(s_parts, axis=0)
    g_sc[i] = jnp.concatenate(g_parts, axis=0)

    @pl.when(i == nsteps - 1)
    def _fin():
        m = m_sc[...]
        mm = jnp.max(m, axis=(0, 1), keepdims=True)
        s = jnp.sum(s_sc[...] * jnp.exp(m - mm), axis=(0, 1),
                    keepdims=True)
        lse_ref[...] = (jnp.log(s) + mm).reshape(1, _N)
        g_ref[...] = jnp.sum(g_sc[...], axis=(0, 1), keepdims=True
                             ).reshape(1, _N)


def _combine_kernel(t_ref, cw_ref, cb_ref, hpt_ref, lh_ref, gh_ref,
                    l1_ref, g1_ref, l2_ref, g2_ref, o_ref):
    t = t_ref[...]
    cl = jax.lax.dot_general(
        cw_ref[...], hpt_ref[...], (((1,), (0,)), ((), ())),
        preferred_element_type=jnp.float32) + cb_ref[...]
    cl0 = cl[0:1, :]
    cl1 = cl[1:2, :]
    lh = lh_ref[...]
    m = jnp.maximum(jnp.maximum(lh, cl0), cl1)
    lse = m + jnp.log(jnp.exp(lh - m) + jnp.exp(cl0 - m) + jnp.exp(cl1 - m))
    in1 = (t >= _CUT1) & (t < _CUT2)
    in2 = t >= _CUT2
    g = jnp.where(in1, cl1, jnp.where(in2, cl0, gh_ref[...]))
    nll = lse - g
    nll = nll + jnp.where(in1, l1_ref[...] - g1_ref[...], 0.0)
    nll = nll + jnp.where(in2, l2_ref[...] - g2_ref[...], 0.0)
    o_ref[...] = nll


def _stream_lse(hpt, w, b, idx, tile, chunk):
    v, d = w.shape
    nsteps = (v + tile - 1) // tile
    nch = tile // chunk
    vec = jax.ShapeDtypeStruct((1, _N), jnp.float32)
    full = pl.BlockSpec((1, _N), lambda i: (0, 0))
    part = pltpu.VMEM((nsteps, nch, _N), jnp.float32)
    return pl.pallas_call(
        functools.partial(_lse_kernel, tile=tile, chunk=chunk,
                          nsteps=nsteps, nvalid=v),
        grid=(nsteps,),
        in_specs=[
            pl.BlockSpec((1, _N), lambda i: (0, 0)),
            pl.BlockSpec((d, _N), lambda i: (0, 0)),
            pl.BlockSpec((tile, d), lambda i: (i, 0)),
            pl.BlockSpec((tile, 1), lambda i: (i, 0)),
        ],
        out_specs=[full, full],
        out_shape=[vec, vec],
        scratch_shapes=[
            pltpu.VMEM((chunk, _N), jnp.int32), part, part, part,
        ],
    )(idx, hpt, w, b)


def kernel(hidden, target, cluster_weight, cluster_bias, proj0, proj1,
           proj2, w0, w1, w2, b0, b1, b2):
    bf = jnp.bfloat16

    pt = jnp.concatenate([proj0, proj1, proj2], axis=1).T.astype(bf)
    ht = hidden.T.astype(bf)

    hpt = pl.pallas_call(
        _proj_kernel,
        out_shape=jax.ShapeDtypeStruct((pt.shape[0], _N), bf),
    )(pt, ht)
    hpt0 = hpt[:_D]
    hpt1 = hpt[_D:_D + 256]
    hpt2 = hpt[_D + 256:_D + 320]

    t = target.astype(jnp.int32).reshape(1, _N)
    idx_1 = t - _CUT1
    idx_2 = t - _CUT2

    lse_h, g_h = _stream_lse(hpt0, w0.astype(jnp.float32),
                             b0.reshape(-1, 1), t, 2048, 512)
    lse_1, g_1 = _stream_lse(hpt1, w1.astype(jnp.float32),
                             b1.reshape(-1, 1), idx_1, 2048, 512)
    lse_2, g_2 = _stream_lse(hpt2, w2.astype(jnp.float32),
                             b2.reshape(-1, 1), idx_2, 4096, 512)

    nll = pl.pallas_call(
        _combine_kernel,
        out_shape=jax.ShapeDtypeStruct((1, _N), jnp.float32),
    )(t, cluster_weight.astype(bf), cluster_bias.reshape(2, 1), hpt0,
      lse_h, g_h, lse_1, g_1, lse_2, g_2)
    return nll.reshape(_N)


# speedup vs baseline: 2.4586x; 1.1480x over previous
import functools

import jax
import jax.numpy as jnp
from jax.experimental import pallas as pl
from jax.experimental.pallas import tpu as pltpu

_N = 2048
_D = 1024
_CUT1 = 20000
_CUT2 = 40000
_NEG = -1e30


def _proj_kernel(p_ref, h_ref, o_ref):
    o_ref[...] = jax.lax.dot_general(
        p_ref[...], h_ref[...], (((1,), (0,)), ((), ())),
        preferred_element_type=jnp.float32).astype(jnp.bfloat16)


def _lse_kernel(idx_ref, hpt_ref, w_ref, b_ref, lse_ref, g_ref,
                iota_sc, m_sc, s_sc, g_sc, *, tile, chunk, nsteps, nvalid):
    i = pl.program_id(0)
    nch = tile // chunk

    @pl.when(i == 0)
    def _init():
        iota_sc[...] = jax.lax.broadcasted_iota(
            jnp.int32, iota_sc.shape, 0)

    idx = idx_ref[...]
    m_parts, s_parts, g_parts = [], [], []
    for c in range(nch):
        rows = pl.ds(c * chunk, chunk)
        lt = jax.lax.dot_general(
            w_ref[rows, :].astype(jnp.bfloat16), hpt_ref[...],
            (((1,), (0,)), ((), ())),
            preferred_element_type=jnp.float32).astype(jnp.bfloat16)
        lt = lt + b_ref[rows, :].astype(jnp.bfloat16)
        hit = iota_sc[...] == (idx - (i * tile + c * chunk))
        g_parts.append(jnp.sum(jnp.where(hit, lt, jnp.bfloat16(0)),
                               axis=0, keepdims=True, dtype=jnp.float32))
        m_c = jnp.max(lt, axis=0, keepdims=True)
        p = jnp.exp(lt - m_c)
        s_parts.append(jnp.sum(p, axis=0, keepdims=True,
                               dtype=jnp.float32))
        m_parts.append(m_c.astype(jnp.float32))
    m_sc[i] = jnp.concatenate(m_parts, axis=0)
    s_sc[i] = jnp.concatenate(s_parts, axis=0)
    g_sc[i] = jnp.concatenate(g_parts, axis=0)

    @pl.when(i == nsteps - 1)
    def _fin():
        m = m_sc[...]
        mm = jnp.max(m, axis=(0, 1), keepdims=True)
        s = jnp.sum(s_sc[...] * jnp.exp(m - mm), axis=(0, 1),
                    keepdims=True)
        lse_ref[...] = (jnp.log(s) + mm).reshape(1, _N)
        g_ref[...] = jnp.sum(g_sc[...], axis=(0, 1), keepdims=True
                             ).reshape(1, _N)


def _combine_kernel(t_ref, cw_ref, cb_ref, hpt_ref, lh_ref, gh_ref,
                    l1_ref, g1_ref, l2_ref, g2_ref, o_ref):
    t = t_ref[...]
    cl = jax.lax.dot_general(
        cw_ref[...], hpt_ref[...], (((1,), (0,)), ((), ())),
        preferred_element_type=jnp.float32) + cb_ref[...]
    cl0 = cl[0:1, :]
    cl1 = cl[1:2, :]
    lh = lh_ref[...]
    m = jnp.maximum(jnp.maximum(lh, cl0), cl1)
    lse = m + jnp.log(jnp.exp(lh - m) + jnp.exp(cl0 - m) + jnp.exp(cl1 - m))
    in1 = (t >= _CUT1) & (t < _CUT2)
    in2 = t >= _CUT2
    g = jnp.where(in1, cl1, jnp.where(in2, cl0, gh_ref[...]))
    nll = lse - g
    nll = nll + jnp.where(in1, l1_ref[...] - g1_ref[...], 0.0)
    nll = nll + jnp.where(in2, l2_ref[...] - g2_ref[...], 0.0)
    o_ref[...] = nll


def _stream_lse(hpt, w, b, idx, tile, chunk):
    v, d = w.shape
    nsteps = (v + tile - 1) // tile
    nch = tile // chunk
    vec = jax.ShapeDtypeStruct((1, _N), jnp.float32)
    full = pl.BlockSpec((1, _N), lambda i: (0, 0))
    part = pltpu.VMEM((nsteps, nch, _N), jnp.float32)
    return pl.pallas_call(
        functools.partial(_lse_kernel, tile=tile, chunk=chunk,
                          nsteps=nsteps, nvalid=v),
        grid=(nsteps,),
        in_specs=[
            pl.BlockSpec((1, _N), lambda i: (0, 0)),
            pl.BlockSpec((d, _N), lambda i: (0, 0)),
            pl.BlockSpec((tile, d), lambda i: (i, 0)),
            pl.BlockSpec((tile, 1), lambda i: (i, 0)),
        ],
        out_specs=[full, full],
        out_shape=[vec, vec],
        scratch_shapes=[
            pltpu.VMEM((chunk, _N), jnp.int32), part, part, part,
        ],
    )(idx, hpt, w, b)


def kernel(hidden, target, cluster_weight, cluster_bias, proj0, proj1,
           proj2, w0, w1, w2, b0, b1, b2):
    bf = jnp.bfloat16

    pt = jnp.concatenate([proj0, proj1, proj2], axis=1).T.astype(bf)
    ht = hidden.T.astype(bf)

    hpt = pl.pallas_call(
        _proj_kernel,
        out_shape=jax.ShapeDtypeStruct((pt.shape[0], _N), bf),
    )(pt, ht)
    hpt0 = hpt[:_D]
    hpt1 = hpt[_D:_D + 256]
    hpt2 = hpt[_D + 256:_D + 320]

    t = target.astype(jnp.int32).reshape(1, _N)
    idx_1 = t - _CUT1
    idx_2 = t - _CUT2

    lse_h, g_h = _stream_lse(hpt0, w0.astype(jnp.float32),
                             b0.reshape(-1, 1), t, 2000, 400)
    lse_1, g_1 = _stream_lse(hpt1, w1.astype(jnp.float32),
                             b1.reshape(-1, 1), idx_1, 2000, 400)
    lse_2, g_2 = _stream_lse(hpt2, w2.astype(jnp.float32),
                             b2.reshape(-1, 1), idx_2, 4000, 400)

    nll = pl.pallas_call(
        _combine_kernel,
        out_shape=jax.ShapeDtypeStruct((1, _N), jnp.float32),
    )(t, cluster_weight.astype(bf), cluster_bias.reshape(2, 1), hpt0,
      lse_h, g_h, lse_1, g_1, lse_2, g_2)
    return nll.reshape(_N)
